# baseline (speedup 1.0000x reference)
import functools

import jax
import jax.numpy as jnp
from jax import lax
from jax.experimental import pallas as pl
from jax.experimental.pallas import tpu as pltpu
from jax.experimental.pallas import tpu_sc as plsc

D = 256
W = D // 2
E = 160000
NC = 2
NS = 16
NW = NC * NS
L = 16
EW = E // NW
C = 128
NCH = -(-EW // C)
LAST_BASE = EW - C


def _sc_sqdist(zi, ei):
    mesh = plsc.VectorSubcoreMesh(core_axis_name="c", subcore_axis_name="s")

    @functools.partial(
        pl.kernel,
        out_type=jax.ShapeDtypeStruct((E,), jnp.float32),
        mesh=mesh,
        scratch_types=[
            pltpu.VMEM((EW,), jnp.int32),
            pltpu.VMEM((EW,), jnp.int32),
            pltpu.VMEM((C, W), jnp.int32),
            pltpu.VMEM((C, W), jnp.int32),
            pltpu.VMEM((C, W), jnp.int32),
            pltpu.VMEM((C, W), jnp.int32),
            pltpu.VMEM((EW,), jnp.float32),
            pltpu.SemaphoreType.DMA,
            pltpu.SemaphoreType.DMA,
            pltpu.SemaphoreType.DMA,
            pltpu.SemaphoreType.DMA,
        ],
        compiler_params=pltpu.CompilerParams(
            use_tc_tiling_on_sc=False, needs_layout_passes=False),
    )
    def k(z_hbm, ei_hbm, out_hbm,
          sidx, didx, sA, dA, sB, dB, qv, sem_sA, sem_dA, sem_sB, sem_dB):
        wid = lax.axis_index("s") * NC + lax.axis_index("c")
        ebase = wid * EW
        pltpu.sync_copy(ei_hbm.at[0, pl.ds(ebase, EW)], sidx)
        pltpu.sync_copy(ei_hbm.at[1, pl.ds(ebase, EW)], didx)

        def chunk_base(c):
            return jnp.minimum(c * C, LAST_BASE)

        def issue(c, s_buf, d_buf, sem_s, sem_d):
            b = chunk_base(c)
            pltpu.async_copy(z_hbm.at[sidx.at[pl.ds(b, C)]], s_buf, sem_s)
            pltpu.async_copy(z_hbm.at[didx.at[pl.ds(b, C)]], d_buf, sem_d)

        def drain(s_buf, d_buf, sem_s, sem_d):
            pltpu.make_async_copy(z_hbm.at[sidx.at[pl.ds(0, C)]],
                                  s_buf, sem_s).wait()
            pltpu.make_async_copy(z_hbm.at[didx.at[pl.ds(0, C)]],
                                  d_buf, sem_d).wait()

        zero16 = jnp.zeros((L,), jnp.int32)
        epsb = jnp.bfloat16(1e-6)
        rot0 = lax.iota(jnp.int32, L)

        def compute(c, s_buf, d_buf):
            qb = chunk_base(c)

            def eb_body(eb, carry):
                flat0 = (rot0 + eb * L) * W

                def grp(_, gc):
                    acc, flatbase = gc
                    for t in range(2 * L):
                        colr = (rot0 + t) & (L - 1)
                        flat = flatbase + (t & ~(L - 1)) + colr
                        a = plsc.bitcast(
                            plsc.load_gather(s_buf, [zero16, flat]),
                            jnp.bfloat16)
                        bb = plsc.bitcast(
                            plsc.load_gather(d_buf, [zero16, flat]),
                            jnp.bfloat16)
                        dlt = a - bb + epsb
                        acc = acc + dlt * dlt
                    return acc, flatbase + 2 * L

                acc, _ = lax.fori_loop(
                    0, W // (2 * L), grp,
                    (jnp.zeros((2 * L,), jnp.bfloat16), flat0))
                lo, hi = plsc.unpack(acc, format=plsc.PackFormat.INTERLEAVED)
                qv[pl.ds(qb + eb * L, L)] = lo + hi
                return carry

            lax.fori_loop(0, C // L, eb_body, 0)

        issue(0, sA, dA, sem_sA, sem_dA)
        issue(1, sB, dB, sem_sB, sem_dB)

        def body(i2, carry):
            c0 = i2 * 2
            c1 = c0 + 1
            drain(sA, dA, sem_sA, sem_dA)
            compute(c0, sA, dA)
            issue(c0 + 2, sA, dA, sem_sA, sem_dA)
            drain(sB, dB, sem_sB, sem_dB)
            compute(c1, sB, dB)
            issue(c1 + 2, sB, dB, sem_sB, sem_dB)
            return carry

        lax.fori_loop(0, (NCH - 2) // 2, body, 0)
        drain(sA, dA, sem_sA, sem_dA)
        compute(NCH - 2, sA, dA)
        drain(sB, dB, sem_sB, sem_dB)
        compute(NCH - 1, sB, dB)
        pltpu.sync_copy(qv, out_hbm.at[pl.ds(ebase, EW)])

    return k(zi, ei)


def _tc_pack(z):
    def body(z_ref, o_ref):
        a = z_ref[:, :W].astype(jnp.bfloat16)
        b = z_ref[:, W:].astype(jnp.bfloat16)
        au = lax.bitcast_convert_type(a, jnp.uint16).astype(jnp.uint32)
        bu = lax.bitcast_convert_type(b, jnp.uint16).astype(jnp.uint32)
        o_ref[...] = lax.bitcast_convert_type(au | (bu << 16), jnp.int32)

    n = z.shape[0]
    nb = 10
    rb = n // nb
    return pl.pallas_call(
        body,
        grid=(nb,),
        in_specs=[pl.BlockSpec((rb, D), lambda i: (i, 0))],
        out_specs=pl.BlockSpec((rb, W), lambda i: (i, 0)),
        out_shape=jax.ShapeDtypeStruct((n, W), jnp.int32),
    )(z)


def _tc_finalize(q):
    rows = E // 128

    def body(q_ref, o_ref):
        qv = q_ref[...]
        d = lax.rsqrt(qv)
        m = jnp.max(d)
        o_ref[...] = jnp.exp(d - m)

    out = pl.pallas_call(
        body,
        out_shape=jax.ShapeDtypeStruct((rows, 128), jnp.float32),
    )(q.reshape(rows, 128))
    return out.reshape(E)


def kernel(z, edge_index, p):
    ei = jnp.asarray(edge_index, jnp.int32)
    zi = _tc_pack(z)
    q = _sc_sqdist(zi, ei)
    return _tc_finalize(q)


# speedup vs baseline: 1.0100x; 1.0100x over previous
import functools

import jax
import jax.numpy as jnp
from jax import lax
from jax.experimental import pallas as pl
from jax.experimental.pallas import tpu as pltpu
from jax.experimental.pallas import tpu_sc as plsc

D = 256
W = D // 2
E = 160000
NC = 2
NS = 16
NW = NC * NS
L = 16
EW = E // NW
C = 128
NCH = -(-EW // C)
LAST_BASE = EW - C


def _sc_sqdist(zi, ei):
    mesh = plsc.VectorSubcoreMesh(core_axis_name="c", subcore_axis_name="s")

    @functools.partial(
        pl.kernel,
        out_type=jax.ShapeDtypeStruct((E,), jnp.float32),
        mesh=mesh,
        scratch_types=[
            pltpu.VMEM((EW,), jnp.int32),
            pltpu.VMEM((EW,), jnp.int32),
            pltpu.VMEM((C, W), jnp.int32),
            pltpu.VMEM((C, W), jnp.int32),
            pltpu.VMEM((C, W), jnp.int32),
            pltpu.VMEM((C, W), jnp.int32),
            pltpu.VMEM((EW,), jnp.float32),
            pltpu.SemaphoreType.DMA,
            pltpu.SemaphoreType.DMA,
            pltpu.SemaphoreType.DMA,
            pltpu.SemaphoreType.DMA,
        ],
        compiler_params=pltpu.CompilerParams(
            use_tc_tiling_on_sc=False, needs_layout_passes=False),
    )
    def k(z_hbm, ei_hbm, out_hbm,
          sidx, didx, sA, dA, sB, dB, qv, sem_sA, sem_dA, sem_sB, sem_dB):
        wid = lax.axis_index("s") * NC + lax.axis_index("c")
        ebase = wid * EW
        pltpu.sync_copy(ei_hbm.at[0, pl.ds(ebase, EW)], sidx)
        pltpu.sync_copy(ei_hbm.at[1, pl.ds(ebase, EW)], didx)

        def chunk_base(c):
            return jnp.minimum(c * C, LAST_BASE)

        def issue(c, s_buf, d_buf, sem_s, sem_d):
            b = chunk_base(c)
            pltpu.async_copy(z_hbm.at[sidx.at[pl.ds(b, C)]], s_buf, sem_s)
            pltpu.async_copy(z_hbm.at[didx.at[pl.ds(b, C)]], d_buf, sem_d)

        def drain(s_buf, d_buf, sem_s, sem_d):
            pltpu.make_async_copy(z_hbm.at[sidx.at[pl.ds(0, C)]],
                                  s_buf, sem_s).wait()
            pltpu.make_async_copy(z_hbm.at[didx.at[pl.ds(0, C)]],
                                  d_buf, sem_d).wait()

        zero16 = jnp.zeros((L,), jnp.int32)
        epsb = jnp.bfloat16(1e-6)
        rot0 = lax.iota(jnp.int32, L)

        def compute(c, s_buf, d_buf):
            qb = chunk_base(c)

            def eb_body(eb, carry):
                flat0 = (rot0 + eb * L) * W

                def grp(_, gc):
                    acc, flatbase = gc
                    for t in range(L):
                        colr = (rot0 + t) & (L - 1)
                        flat = flatbase + colr
                        a = plsc.bitcast(
                            plsc.load_gather(s_buf, [zero16, flat]),
                            jnp.bfloat16)
                        bb = plsc.bitcast(
                            plsc.load_gather(d_buf, [zero16, flat]),
                            jnp.bfloat16)
                        dlt = a - bb + epsb
                        acc = acc + dlt * dlt
                    return acc, flatbase + L

                acc, _ = lax.fori_loop(
                    0, W // L, grp,
                    (jnp.zeros((2 * L,), jnp.bfloat16), flat0))
                lo, hi = plsc.unpack(acc, format=plsc.PackFormat.INTERLEAVED)
                qv[pl.ds(qb + eb * L, L)] = lo + hi
                return carry

            lax.fori_loop(0, C // L, eb_body, 0)

        issue(0, sA, dA, sem_sA, sem_dA)
        issue(1, sB, dB, sem_sB, sem_dB)

        def body(i2, carry):
            c0 = i2 * 2
            c1 = c0 + 1
            drain(sA, dA, sem_sA, sem_dA)
            compute(c0, sA, dA)
            issue(c0 + 2, sA, dA, sem_sA, sem_dA)
            drain(sB, dB, sem_sB, sem_dB)
            compute(c1, sB, dB)
            issue(c1 + 2, sB, dB, sem_sB, sem_dB)
            return carry

        lax.fori_loop(0, (NCH - 2) // 2, body, 0)
        drain(sA, dA, sem_sA, sem_dA)
        compute(NCH - 2, sA, dA)
        drain(sB, dB, sem_sB, sem_dB)
        compute(NCH - 1, sB, dB)
        pltpu.sync_copy(qv, out_hbm.at[pl.ds(ebase, EW)])

    return k(zi, ei)


def _tc_pack(z):
    def body(z_ref, o_ref):
        a = z_ref[:, :W].astype(jnp.bfloat16)
        b = z_ref[:, W:].astype(jnp.bfloat16)
        au = lax.bitcast_convert_type(a, jnp.uint16).astype(jnp.uint32)
        bu = lax.bitcast_convert_type(b, jnp.uint16).astype(jnp.uint32)
        o_ref[...] = lax.bitcast_convert_type(au | (bu << 16), jnp.int32)

    n = z.shape[0]
    nb = 10
    rb = n // nb
    return pl.pallas_call(
        body,
        grid=(nb,),
        in_specs=[pl.BlockSpec((rb, D), lambda i: (i, 0))],
        out_specs=pl.BlockSpec((rb, W), lambda i: (i, 0)),
        out_shape=jax.ShapeDtypeStruct((n, W), jnp.int32),
    )(z)


def _tc_finalize(q):
    rows = E // 128

    def body(q_ref, o_ref):
        qv = q_ref[...]
        d = lax.rsqrt(qv)
        m = jnp.max(d)
        o_ref[...] = jnp.exp(d - m)

    out = pl.pallas_call(
        body,
        out_shape=jax.ShapeDtypeStruct((rows, 128), jnp.float32),
    )(q.reshape(rows, 128))
    return out.reshape(E)


def kernel(z, edge_index, p):
    ei = jnp.asarray(edge_index, jnp.int32)
    zi = _tc_pack(z)
    q = _sc_sqdist(zi, ei)
    return _tc_finalize(q)


# speedup vs baseline: 1.0399x; 1.0296x over previous
import functools

import jax
import jax.numpy as jnp
from jax import lax
from jax.experimental import pallas as pl
from jax.experimental.pallas import tpu as pltpu
from jax.experimental.pallas import tpu_sc as plsc

D = 256
W = D // 2
E = 160000
NC = 2
NS = 16
NW = NC * NS
L = 16
EW = E // NW
C = 128
NCH = -(-EW // C)
LAST_BASE = EW - C


def _sc_sqdist(zi, ei):
    mesh = plsc.VectorSubcoreMesh(core_axis_name="c", subcore_axis_name="s")

    @functools.partial(
        pl.kernel,
        out_type=jax.ShapeDtypeStruct((E,), jnp.float32),
        mesh=mesh,
        scratch_types=[
            pltpu.VMEM((EW,), jnp.int32),
            pltpu.VMEM((EW,), jnp.int32),
            pltpu.VMEM((C, W), jnp.int32),
            pltpu.VMEM((C, W), jnp.int32),
            pltpu.VMEM((C, W), jnp.int32),
            pltpu.VMEM((C, W), jnp.int32),
            pltpu.VMEM((EW,), jnp.float32),
            pltpu.SemaphoreType.DMA,
            pltpu.SemaphoreType.DMA,
            pltpu.SemaphoreType.DMA,
            pltpu.SemaphoreType.DMA,
        ],
        compiler_params=pltpu.CompilerParams(
            use_tc_tiling_on_sc=False, needs_layout_passes=False),
    )
    def k(z_hbm, ei_hbm, out_hbm,
          sidx, didx, sA, dA, sB, dB, qv, sem_sA, sem_dA, sem_sB, sem_dB):
        wid = lax.axis_index("s") * NC + lax.axis_index("c")
        ebase = wid * EW
        pltpu.sync_copy(ei_hbm.at[0, pl.ds(ebase, EW)], sidx)
        pltpu.sync_copy(ei_hbm.at[1, pl.ds(ebase, EW)], didx)

        def chunk_base(c):
            return jnp.minimum(c * C, LAST_BASE)

        def issue(c, s_buf, d_buf, sem_s, sem_d):
            b = chunk_base(c)
            pltpu.async_copy(z_hbm.at[sidx.at[pl.ds(b, C)]], s_buf, sem_s)
            pltpu.async_copy(z_hbm.at[didx.at[pl.ds(b, C)]], d_buf, sem_d)

        def drain(s_buf, d_buf, sem_s, sem_d):
            pltpu.make_async_copy(z_hbm.at[sidx.at[pl.ds(0, C)]],
                                  s_buf, sem_s).wait()
            pltpu.make_async_copy(z_hbm.at[didx.at[pl.ds(0, C)]],
                                  d_buf, sem_d).wait()

        zero16 = jnp.zeros((L,), jnp.int32)
        epsb = jnp.bfloat16(1e-6)
        rot0 = lax.iota(jnp.int32, L)

        def compute(c, s_buf, d_buf):
            qb = chunk_base(c)

            def eb_body(eb, carry):
                flat0 = (rot0 + eb * L) * W

                def grp(_, gc):
                    acc, flatbase = gc
                    for t in range(L):
                        colr = (rot0 + t) & (L - 1)
                        flat = flatbase + colr
                        a = plsc.bitcast(
                            plsc.load_gather(s_buf, [zero16, flat]),
                            jnp.bfloat16)
                        bb = plsc.bitcast(
                            plsc.load_gather(d_buf, [zero16, flat]),
                            jnp.bfloat16)
                        dlt = a - bb + epsb
                        acc = acc + dlt * dlt
                    return acc, flatbase + L

                acc, _ = lax.fori_loop(
                    0, W // L, grp,
                    (jnp.zeros((2 * L,), jnp.bfloat16), flat0))
                lo, hi = plsc.unpack(acc, format=plsc.PackFormat.INTERLEAVED)
                qv[pl.ds(qb + eb * L, L)] = lo + hi
                return carry

            lax.fori_loop(0, C // L, eb_body, 0)

        issue(0, sA, dA, sem_sA, sem_dA)
        issue(1, sB, dB, sem_sB, sem_dB)

        def body(i2, carry):
            c0 = i2 * 2
            c1 = c0 + 1
            drain(sA, dA, sem_sA, sem_dA)
            compute(c0, sA, dA)
            issue(c0 + 2, sA, dA, sem_sA, sem_dA)
            drain(sB, dB, sem_sB, sem_dB)
            compute(c1, sB, dB)
            issue(c1 + 2, sB, dB, sem_sB, sem_dB)
            return carry

        lax.fori_loop(0, (NCH - 2) // 2, body, 0)
        drain(sA, dA, sem_sA, sem_dA)
        compute(NCH - 2, sA, dA)
        drain(sB, dB, sem_sB, sem_dB)
        compute(NCH - 1, sB, dB)
        pltpu.sync_copy(qv, out_hbm.at[pl.ds(ebase, EW)])

    return k(zi, ei)


def _tc_pack(z):
    def body(z_ref, o_ref):
        a = z_ref[:, :W].astype(jnp.bfloat16)
        b = z_ref[:, W:].astype(jnp.bfloat16)
        au = lax.bitcast_convert_type(a, jnp.uint16).astype(jnp.uint32)
        bu = lax.bitcast_convert_type(b, jnp.uint16).astype(jnp.uint32)
        o_ref[...] = lax.bitcast_convert_type(au | (bu << 16), jnp.int32)

    return pl.pallas_call(
        body,
        out_shape=jax.ShapeDtypeStruct((z.shape[0], W), jnp.int32),
    )(z)


def _tc_finalize(q):
    rows = E // 128

    def body(q_ref, o_ref):
        qv = q_ref[...]
        d = lax.rsqrt(qv)
        m = jnp.max(d)
        o_ref[...] = jnp.exp(d - m)

    out = pl.pallas_call(
        body,
        out_shape=jax.ShapeDtypeStruct((rows, 128), jnp.float32),
    )(q.reshape(rows, 128))
    return out.reshape(E)


def kernel(z, edge_index, p):
    ei = jnp.asarray(edge_index, jnp.int32)
    zi = _tc_pack(z)
    q = _sc_sqdist(zi, ei)
    return _tc_finalize(q)
